# baseline (speedup 1.0000x reference)
import functools

import jax
import jax.numpy as jnp
from jax import lax
from jax.experimental import pallas as pl
from jax.experimental.pallas import tpu as pltpu
from jax.experimental.pallas import tpu_sc as plsc

_B, _P, _NS = 16, 16384, 1024
_L = 16
_U = 8
_NGRP = _P // _L


def _fps_tile(px_hbm, py_hbm, pz_hbm, oi_hbm, ox_hbm, oy_hbm, oz_hbm,
              px_v, py_v, pz_v, dist_v, oi_v, ox_v, oy_v, oz_v):
    c = lax.axis_index("c")
    s = lax.axis_index("s")
    b = c * 8 + s

    @pl.when(s < 8)
    def _():
        pltpu.sync_copy(px_hbm.at[pl.ds(b * _P, _P)], px_v)
        pltpu.sync_copy(py_hbm.at[pl.ds(b * _P, _P)], py_v)
        pltpu.sync_copy(pz_hbm.at[pl.ds(b * _P, _P)], pz_v)

        lanes = lax.iota(jnp.int32, _L)
        big = jnp.full((_L,), 1e10, jnp.float32)
        neg = jnp.full((_L,), -1e30, jnp.float32)

        def init_body(g, carry):
            dist_v[pl.ds(g * _L, _L)] = big
            return carry

        lax.fori_loop(0, _NGRP, init_body, 0)

        def fetch(ref, gi, lm):
            grp = ref[pl.ds(gi * _L, _L)]
            return jnp.max(jnp.where(lanes == lm, grp, neg))

        zero = jnp.int32(0)
        cx0 = fetch(px_v, zero, zero)
        cy0 = fetch(py_v, zero, zero)
        cz0 = fetch(pz_v, zero, zero)
        zvec = jnp.zeros((_L,), jnp.float32)

        def sample_body(si, carry):
            cur, cx, cy, cz, ai, ax, ay, az = carry
            sl = si % _L
            ai = jnp.where(lanes == sl, cur, ai)
            ax = jnp.where(lanes == sl, cx, ax)
            ay = jnp.where(lanes == sl, cy, ay)
            az = jnp.where(lanes == sl, cz, az)

            @pl.when(sl == _L - 1)
            def _():
                base = (si // _L) * _L
                oi_v[pl.ds(base, _L)] = ai
                ox_v[pl.ds(base, _L)] = ax
                oy_v[pl.ds(base, _L)] = ay
                oz_v[pl.ds(base, _L)] = az

            def g_body(g, gc):
                bv, bi = gc
                gbase = g * (_U * _L)
                for u in range(_U):
                    off = gbase + u * _L
                    px = px_v[pl.ds(off, _L)]
                    py = py_v[pl.ds(off, _L)]
                    pz = pz_v[pl.ds(off, _L)]
                    dx = px - cx
                    dy = py - cy
                    dz = pz - cz
                    d = (dx * dx + dy * dy) + dz * dz
                    dd = jnp.minimum(dist_v[pl.ds(off, _L)], d)
                    dist_v[pl.ds(off, _L)] = dd
                    upd = dd > bv
                    bv = jnp.where(upd, dd, bv)
                    bi = jnp.where(upd, off + lanes, bi)
                return bv, bi

            bv0 = jnp.full((_L,), -1.0, jnp.float32)
            bi0 = jnp.zeros((_L,), jnp.int32)
            bv, bi = lax.fori_loop(0, _NGRP // _U, g_body, (bv0, bi0))

            m = jnp.max(bv)
            cand = jnp.where(bv == m, bi, _P)
            li = jnp.min(cand)
            gi, lm = li // _L, li % _L
            ncx = fetch(px_v, gi, lm)
            ncy = fetch(py_v, gi, lm)
            ncz = fetch(pz_v, gi, lm)
            return li, ncx, ncy, ncz, ai, ax, ay, az

        lax.fori_loop(0, _NS, sample_body,
                      (zero, cx0, cy0, cz0,
                       jnp.zeros((_L,), jnp.int32), zvec, zvec, zvec))

        pltpu.sync_copy(oi_v, oi_hbm.at[pl.ds(b * _NS, _NS)])
        pltpu.sync_copy(ox_v, ox_hbm.at[pl.ds(b * _NS, _NS)])
        pltpu.sync_copy(oy_v, oy_hbm.at[pl.ds(b * _NS, _NS)])
        pltpu.sync_copy(oz_v, oz_hbm.at[pl.ds(b * _NS, _NS)])


@jax.jit
def _run_sc(points):
    pts = jnp.transpose(points, (2, 0, 1))
    px = pts[0].reshape(_B * _P)
    py = pts[1].reshape(_B * _P)
    pz = pts[2].reshape(_B * _P)
    mesh = plsc.VectorSubcoreMesh(core_axis_name="c", subcore_axis_name="s")
    fn = pl.kernel(
        _fps_tile,
        out_type=[
            jax.ShapeDtypeStruct((_B * _NS,), jnp.int32),
            jax.ShapeDtypeStruct((_B * _NS,), jnp.float32),
            jax.ShapeDtypeStruct((_B * _NS,), jnp.float32),
            jax.ShapeDtypeStruct((_B * _NS,), jnp.float32),
        ],
        mesh=mesh,
        compiler_params=pltpu.CompilerParams(needs_layout_passes=False),
        scratch_types=[
            pltpu.VMEM((_P,), jnp.float32),
            pltpu.VMEM((_P,), jnp.float32),
            pltpu.VMEM((_P,), jnp.float32),
            pltpu.VMEM((_P,), jnp.float32),
            pltpu.VMEM((_NS,), jnp.int32),
            pltpu.VMEM((_NS,), jnp.float32),
            pltpu.VMEM((_NS,), jnp.float32),
            pltpu.VMEM((_NS,), jnp.float32),
        ],
    )
    oi, ox, oy, oz = fn(px, py, pz)
    oi = oi.reshape(_B, _NS)
    sampled = jnp.stack([ox.reshape(_B, _NS), oy.reshape(_B, _NS),
                         oz.reshape(_B, _NS)], axis=-1)
    return oi, sampled


def kernel(points, nsamples, kd_depth, return_gathered):
    return _run_sc(points)


# speedup vs baseline: 2.9442x; 2.9442x over previous
import functools

import jax
import jax.numpy as jnp
from jax import lax
from jax.experimental import pallas as pl
from jax.experimental.pallas import tpu as pltpu
from jax.experimental.pallas import tpu_sc as plsc

_B, _P, _NS = 16, 16384, 1024
_L = 16
_U = 8
_NGRP = _P // _L


def _fps_tile(px_hbm, py_hbm, pz_hbm, oi_hbm, ox_hbm, oy_hbm, oz_hbm,
              px_v, py_v, pz_v, dist_v, oi_v, ox_v, oy_v, oz_v):
    c = lax.axis_index("c")
    s = lax.axis_index("s")
    b = c * 8 + s

    @pl.when(s < 8)
    def _():
        pltpu.sync_copy(px_hbm.at[pl.ds(b * _P, _P)], px_v)
        pltpu.sync_copy(py_hbm.at[pl.ds(b * _P, _P)], py_v)
        pltpu.sync_copy(pz_hbm.at[pl.ds(b * _P, _P)], pz_v)

        lanes = lax.iota(jnp.int32, _L)
        big = jnp.full((_L,), 1e10, jnp.float32)
        neg = jnp.full((_L,), -1e30, jnp.float32)

        def init_body(g, carry):
            dist_v[pl.ds(g * _L, _L)] = big
            return carry

        lax.fori_loop(0, _NGRP, init_body, 0)

        def fetch(ref, gi, lm):
            grp = ref[pl.ds(gi * _L, _L)]
            return jnp.max(jnp.where(lanes == lm, grp, neg))

        zero = jnp.int32(0)
        cx0 = fetch(px_v, zero, zero)
        cy0 = fetch(py_v, zero, zero)
        cz0 = fetch(pz_v, zero, zero)
        zvec = jnp.zeros((_L,), jnp.float32)

        def sample_body(si, carry):
            cur, cx, cy, cz, ai, ax, ay, az = carry
            sl = si % _L
            ai = jnp.where(lanes == sl, cur, ai)
            ax = jnp.where(lanes == sl, cx, ax)
            ay = jnp.where(lanes == sl, cy, ay)
            az = jnp.where(lanes == sl, cz, az)

            @pl.when(sl == _L - 1)
            def _():
                base = (si // _L) * _L
                oi_v[pl.ds(base, _L)] = ai
                ox_v[pl.ds(base, _L)] = ax
                oy_v[pl.ds(base, _L)] = ay
                oz_v[pl.ds(base, _L)] = az

            def g_body(g, gc):
                bv, bi = gc
                gbase = g * (_U * _L)
                dds = []
                for u in range(_U):
                    off = gbase + u * _L
                    px = px_v[pl.ds(off, _L)]
                    py = py_v[pl.ds(off, _L)]
                    pz = pz_v[pl.ds(off, _L)]
                    dx = px - cx
                    dy = py - cy
                    dz = pz - cz
                    d = (dx * dx + dy * dy) + dz * dz
                    dd = jnp.minimum(dist_v[pl.ds(off, _L)], d)
                    dds.append(dd)
                    upd = dd > bv
                    bv = jnp.where(upd, dd, bv)
                    bi = jnp.where(upd, off + lanes, bi)
                for u in range(_U):
                    dist_v[pl.ds(gbase + u * _L, _L)] = dds[u]
                return bv, bi

            bv0 = jnp.full((_L,), -1.0, jnp.float32)
            bi0 = jnp.zeros((_L,), jnp.int32)
            bv, bi = lax.fori_loop(0, _NGRP // _U, g_body, (bv0, bi0))

            m = jnp.max(bv)
            cand = jnp.where(bv == m, bi, _P)
            li = jnp.min(cand)
            gi, lm = li // _L, li % _L
            ncx = fetch(px_v, gi, lm)
            ncy = fetch(py_v, gi, lm)
            ncz = fetch(pz_v, gi, lm)
            return li, ncx, ncy, ncz, ai, ax, ay, az

        lax.fori_loop(0, _NS, sample_body,
                      (zero, cx0, cy0, cz0,
                       jnp.zeros((_L,), jnp.int32), zvec, zvec, zvec))

        pltpu.sync_copy(oi_v, oi_hbm.at[pl.ds(b * _NS, _NS)])
        pltpu.sync_copy(ox_v, ox_hbm.at[pl.ds(b * _NS, _NS)])
        pltpu.sync_copy(oy_v, oy_hbm.at[pl.ds(b * _NS, _NS)])
        pltpu.sync_copy(oz_v, oz_hbm.at[pl.ds(b * _NS, _NS)])


@jax.jit
def _run_sc(points):
    pts = jnp.transpose(points, (2, 0, 1))
    px = pts[0].reshape(_B * _P)
    py = pts[1].reshape(_B * _P)
    pz = pts[2].reshape(_B * _P)
    mesh = plsc.VectorSubcoreMesh(core_axis_name="c", subcore_axis_name="s")
    fn = pl.kernel(
        _fps_tile,
        out_type=[
            jax.ShapeDtypeStruct((_B * _NS,), jnp.int32),
            jax.ShapeDtypeStruct((_B * _NS,), jnp.float32),
            jax.ShapeDtypeStruct((_B * _NS,), jnp.float32),
            jax.ShapeDtypeStruct((_B * _NS,), jnp.float32),
        ],
        mesh=mesh,
        compiler_params=pltpu.CompilerParams(needs_layout_passes=False),
        scratch_types=[
            pltpu.VMEM((_P,), jnp.float32),
            pltpu.VMEM((_P,), jnp.float32),
            pltpu.VMEM((_P,), jnp.float32),
            pltpu.VMEM((_P,), jnp.float32),
            pltpu.VMEM((_NS,), jnp.int32),
            pltpu.VMEM((_NS,), jnp.float32),
            pltpu.VMEM((_NS,), jnp.float32),
            pltpu.VMEM((_NS,), jnp.float32),
        ],
    )
    oi, ox, oy, oz = fn(px, py, pz)
    oi = oi.reshape(_B, _NS)
    sampled = jnp.stack([ox.reshape(_B, _NS), oy.reshape(_B, _NS),
                         oz.reshape(_B, _NS)], axis=-1)
    return oi, sampled


def kernel(points, nsamples, kd_depth, return_gathered):
    return _run_sc(points)


# speedup vs baseline: 3.1626x; 1.0742x over previous
import functools

import jax
import jax.numpy as jnp
from jax import lax
from jax.experimental import pallas as pl
from jax.experimental.pallas import tpu as pltpu
from jax.experimental.pallas import tpu_sc as plsc

_B, _P, _NS = 16, 16384, 1024
_L = 16
_U = 16
_NGRP = _P // _L


def _fps_tile(px_hbm, py_hbm, pz_hbm, oi_hbm, ox_hbm, oy_hbm, oz_hbm,
              px_v, py_v, pz_v, dist_v, oi_v, ox_v, oy_v, oz_v):
    c = lax.axis_index("c")
    s = lax.axis_index("s")
    b = c * 8 + s

    @pl.when(s < 8)
    def _():
        pltpu.sync_copy(px_hbm.at[pl.ds(b * _P, _P)], px_v)
        pltpu.sync_copy(py_hbm.at[pl.ds(b * _P, _P)], py_v)
        pltpu.sync_copy(pz_hbm.at[pl.ds(b * _P, _P)], pz_v)

        lanes = lax.iota(jnp.int32, _L)
        big = jnp.full((_L,), 1e10, jnp.float32)
        neg = jnp.full((_L,), -1e30, jnp.float32)

        def init_body(g, carry):
            dist_v[pl.ds(g * _L, _L)] = big
            return carry

        lax.fori_loop(0, _NGRP, init_body, 0)

        def fetch(ref, gi, lm):
            grp = ref[pl.ds(gi * _L, _L)]
            return jnp.max(jnp.where(lanes == lm, grp, neg))

        zero = jnp.int32(0)
        cx0 = fetch(px_v, zero, zero)
        cy0 = fetch(py_v, zero, zero)
        cz0 = fetch(pz_v, zero, zero)
        zvec = jnp.zeros((_L,), jnp.float32)

        def sample_body(si, carry):
            cur, cx, cy, cz, ai, ax, ay, az = carry
            sl = si % _L
            ai = jnp.where(lanes == sl, cur, ai)
            ax = jnp.where(lanes == sl, cx, ax)
            ay = jnp.where(lanes == sl, cy, ay)
            az = jnp.where(lanes == sl, cz, az)

            @pl.when(sl == _L - 1)
            def _():
                base = (si // _L) * _L
                oi_v[pl.ds(base, _L)] = ai
                ox_v[pl.ds(base, _L)] = ax
                oy_v[pl.ds(base, _L)] = ay
                oz_v[pl.ds(base, _L)] = az

            def g_body(g, gc):
                bv, bi = gc
                gbase = g * (_U * _L)
                dds = []
                for u in range(_U):
                    off = gbase + u * _L
                    px = px_v[pl.ds(off, _L)]
                    py = py_v[pl.ds(off, _L)]
                    pz = pz_v[pl.ds(off, _L)]
                    dx = px - cx
                    dy = py - cy
                    dz = pz - cz
                    d = (dx * dx + dy * dy) + dz * dz
                    dd = jnp.minimum(dist_v[pl.ds(off, _L)], d)
                    dds.append(dd)
                    upd = dd > bv
                    bv = jnp.where(upd, dd, bv)
                    bi = jnp.where(upd, off + lanes, bi)
                for u in range(_U):
                    dist_v[pl.ds(gbase + u * _L, _L)] = dds[u]
                return bv, bi

            bv0 = jnp.full((_L,), -1.0, jnp.float32)
            bi0 = jnp.zeros((_L,), jnp.int32)
            bv, bi = lax.fori_loop(0, _NGRP // _U, g_body, (bv0, bi0))

            m = jnp.max(bv)
            cand = jnp.where(bv == m, bi, _P)
            li = jnp.min(cand)
            gi, lm = li // _L, li % _L
            ncx = fetch(px_v, gi, lm)
            ncy = fetch(py_v, gi, lm)
            ncz = fetch(pz_v, gi, lm)
            return li, ncx, ncy, ncz, ai, ax, ay, az

        lax.fori_loop(0, _NS, sample_body,
                      (zero, cx0, cy0, cz0,
                       jnp.zeros((_L,), jnp.int32), zvec, zvec, zvec))

        pltpu.sync_copy(oi_v, oi_hbm.at[pl.ds(b * _NS, _NS)])
        pltpu.sync_copy(ox_v, ox_hbm.at[pl.ds(b * _NS, _NS)])
        pltpu.sync_copy(oy_v, oy_hbm.at[pl.ds(b * _NS, _NS)])
        pltpu.sync_copy(oz_v, oz_hbm.at[pl.ds(b * _NS, _NS)])


@jax.jit
def _run_sc(points):
    pts = jnp.transpose(points, (2, 0, 1))
    px = pts[0].reshape(_B * _P)
    py = pts[1].reshape(_B * _P)
    pz = pts[2].reshape(_B * _P)
    mesh = plsc.VectorSubcoreMesh(core_axis_name="c", subcore_axis_name="s")
    fn = pl.kernel(
        _fps_tile,
        out_type=[
            jax.ShapeDtypeStruct((_B * _NS,), jnp.int32),
            jax.ShapeDtypeStruct((_B * _NS,), jnp.float32),
            jax.ShapeDtypeStruct((_B * _NS,), jnp.float32),
            jax.ShapeDtypeStruct((_B * _NS,), jnp.float32),
        ],
        mesh=mesh,
        compiler_params=pltpu.CompilerParams(needs_layout_passes=False),
        scratch_types=[
            pltpu.VMEM((_P,), jnp.float32),
            pltpu.VMEM((_P,), jnp.float32),
            pltpu.VMEM((_P,), jnp.float32),
            pltpu.VMEM((_P,), jnp.float32),
            pltpu.VMEM((_NS,), jnp.int32),
            pltpu.VMEM((_NS,), jnp.float32),
            pltpu.VMEM((_NS,), jnp.float32),
            pltpu.VMEM((_NS,), jnp.float32),
        ],
    )
    oi, ox, oy, oz = fn(px, py, pz)
    oi = oi.reshape(_B, _NS)
    sampled = jnp.stack([ox.reshape(_B, _NS), oy.reshape(_B, _NS),
                         oz.reshape(_B, _NS)], axis=-1)
    return oi, sampled


def kernel(points, nsamples, kd_depth, return_gathered):
    return _run_sc(points)


# speedup vs baseline: 11.1351x; 3.5209x over previous
import jax
import jax.numpy as jnp
from jax.experimental import pallas as pl
from jax.experimental.pallas import tpu as pltpu

_B, _P, _NS = 16, 16384, 1024
_C = 128
_NCH = _P // _C


def _fps_body(pts_ref, idx_ref, sx_ref, sy_ref, sz_ref, dist_ref):
    dist_ref[...] = jnp.full((_B, _P), 1e10, jnp.float32)

    nxt0 = jnp.zeros((_B, 1), jnp.int32)
    cx0 = pts_ref[0, :, 0:1]
    cy0 = pts_ref[1, :, 0:1]
    cz0 = pts_ref[2, :, 0:1]

    def body(s, carry):
        nxt, cx, cy, cz = carry
        idx_ref[pl.ds(s, 1), :] = nxt.reshape(1, _B)
        sx_ref[pl.ds(s, 1), :] = cx.reshape(1, _B)
        sy_ref[pl.ds(s, 1), :] = cy.reshape(1, _B)
        sz_ref[pl.ds(s, 1), :] = cz.reshape(1, _B)

        macc = jnp.full((_B, _C), -1.0, jnp.float32)
        chacc = jnp.zeros((_B, _C), jnp.int32)
        xv = jnp.zeros((_B, _C), jnp.float32)
        yv = jnp.zeros((_B, _C), jnp.float32)
        zv = jnp.zeros((_B, _C), jnp.float32)
        for c in range(_NCH):
            o = c * _C
            px = pts_ref[0, :, o:o + _C]
            py = pts_ref[1, :, o:o + _C]
            pz = pts_ref[2, :, o:o + _C]
            dx = px - cx
            dy = py - cy
            dz = pz - cz
            d = (dx * dx + dz * dz) + dy * dy
            dd = jnp.minimum(dist_ref[:, o:o + _C], d)
            dist_ref[:, o:o + _C] = dd
            upd = dd > macc
            macc = jnp.where(upd, dd, macc)
            chacc = jnp.where(upd, c, chacc)
            xv = jnp.where(upd, px, xv)
            yv = jnp.where(upd, py, yv)
            zv = jnp.where(upd, pz, zv)

        base_iota = jax.lax.broadcasted_iota(jnp.int32, (_B, _C), 1)
        candidx = chacc * _C + base_iota
        m = jnp.max(macc, axis=1, keepdims=True)
        nxt2 = jnp.min(jnp.where(macc == m, candidx, _P),
                       axis=1, keepdims=True)
        sel = candidx == nxt2
        cx2 = jnp.max(jnp.where(sel, xv, -1e30), axis=1, keepdims=True)
        cy2 = jnp.max(jnp.where(sel, yv, -1e30), axis=1, keepdims=True)
        cz2 = jnp.max(jnp.where(sel, zv, -1e30), axis=1, keepdims=True)
        return nxt2, cx2, cy2, cz2

    jax.lax.fori_loop(0, _NS, body, (nxt0, cx0, cy0, cz0))


def _run(points, interpret=False):
    pts = jnp.transpose(points, (2, 0, 1))
    idx_t, sx, sy, sz = pl.pallas_call(
        _fps_body,
        out_shape=[
            jax.ShapeDtypeStruct((_NS, _B), jnp.int32),
            jax.ShapeDtypeStruct((_NS, _B), jnp.float32),
            jax.ShapeDtypeStruct((_NS, _B), jnp.float32),
            jax.ShapeDtypeStruct((_NS, _B), jnp.float32),
        ],
        scratch_shapes=[pltpu.VMEM((_B, _P), jnp.float32)],
        interpret=interpret,
    )(pts)
    indices = jnp.transpose(idx_t)
    sampled = jnp.stack([jnp.transpose(sx), jnp.transpose(sy),
                         jnp.transpose(sz)], axis=-1)
    return indices, sampled


def kernel(points, nsamples, kd_depth, return_gathered):
    return _run(points)


# speedup vs baseline: 11.6549x; 1.0467x over previous
import jax
import jax.numpy as jnp
from jax.experimental import pallas as pl
from jax.experimental.pallas import tpu as pltpu

_B, _P, _NS = 16, 16384, 1024
_C = 128
_NCH = _P // _C


def _fps_body(pts_ref, idx_ref, sx_ref, sy_ref, sz_ref, dist_ref):
    dist_ref[...] = jnp.full((_B, _P), 1e10, jnp.float32)

    nxt0 = jnp.zeros((_B, 1), jnp.int32)
    cx0 = pts_ref[0, :, 0:1]
    cy0 = pts_ref[1, :, 0:1]
    cz0 = pts_ref[2, :, 0:1]

    def body(s, carry):
        nxt, cx, cy, cz = carry
        idx_ref[pl.ds(s, 1), :] = nxt.reshape(1, _B)
        sx_ref[pl.ds(s, 1), :] = cx.reshape(1, _B)
        sy_ref[pl.ds(s, 1), :] = cy.reshape(1, _B)
        sz_ref[pl.ds(s, 1), :] = cz.reshape(1, _B)

        macc = jnp.full((_B, _C), -1.0, jnp.float32)
        chacc = jnp.zeros((_B, _C), jnp.int32)
        xv = jnp.zeros((_B, _C), jnp.float32)
        yv = jnp.zeros((_B, _C), jnp.float32)
        zv = jnp.zeros((_B, _C), jnp.float32)
        for c in range(_NCH):
            o = c * _C
            px = pts_ref[0, :, o:o + _C]
            py = pts_ref[1, :, o:o + _C]
            pz = pts_ref[2, :, o:o + _C]
            dx = px - cx
            dy = py - cy
            dz = pz - cz
            d = (dx * dx + dz * dz) + dy * dy
            dd = jnp.minimum(dist_ref[:, o:o + _C], d)
            dist_ref[:, o:o + _C] = dd
            upd = dd > macc
            macc = jnp.where(upd, dd, macc)
            chacc = jnp.where(upd, c, chacc)
            xv = jnp.where(upd, px, xv)
            yv = jnp.where(upd, py, yv)
            zv = jnp.where(upd, pz, zv)

        base_iota = jax.lax.broadcasted_iota(jnp.int32, (_B, _C), 1)
        candidx = chacc * _C + base_iota
        m = jnp.max(macc, axis=1, keepdims=True)
        nxt2 = jnp.min(jnp.where(macc == m, candidx, _P),
                       axis=1, keepdims=True)
        sel = candidx == nxt2
        cx2 = jnp.max(jnp.where(sel, xv, -1e30), axis=1, keepdims=True)
        cy2 = jnp.max(jnp.where(sel, yv, -1e30), axis=1, keepdims=True)
        cz2 = jnp.max(jnp.where(sel, zv, -1e30), axis=1, keepdims=True)
        return nxt2, cx2, cy2, cz2

    jax.lax.fori_loop(0, _NS, body, (nxt0, cx0, cy0, cz0), unroll=2)


def _run(points, interpret=False):
    pts = jnp.transpose(points, (2, 0, 1))
    idx_t, sx, sy, sz = pl.pallas_call(
        _fps_body,
        out_shape=[
            jax.ShapeDtypeStruct((_NS, _B), jnp.int32),
            jax.ShapeDtypeStruct((_NS, _B), jnp.float32),
            jax.ShapeDtypeStruct((_NS, _B), jnp.float32),
            jax.ShapeDtypeStruct((_NS, _B), jnp.float32),
        ],
        scratch_shapes=[pltpu.VMEM((_B, _P), jnp.float32)],
        interpret=interpret,
    )(pts)
    indices = jnp.transpose(idx_t)
    sampled = jnp.stack([jnp.transpose(sx), jnp.transpose(sy),
                         jnp.transpose(sz)], axis=-1)
    return indices, sampled


def kernel(points, nsamples, kd_depth, return_gathered):
    return _run(points)
